# baseline (speedup 1.0000x reference)
import functools
import math

import jax
import jax.numpy as jnp
from jax.experimental import pallas as pl
from jax.experimental.pallas import tpu as pltpu

_TEMP = math.e
_LANES = 128


def _routing_body(x_ref, gw_ref, gb_ref, coef_ref, xb_ref, *, n_active, n_camp_a):
    scores = jnp.dot(x_ref[...], gw_ref[...], preferred_element_type=jnp.float32)
    scores = (scores + gb_ref[...]) * (1.0 / _TEMP)
    m = jnp.max(scores, axis=-1, keepdims=True)
    ex = jnp.exp(scores - m)
    probs = ex / jnp.sum(ex, axis=-1, keepdims=True)

    lane = jax.lax.broadcasted_iota(jnp.int32, scores.shape, 1)
    rank = jnp.zeros(scores.shape, jnp.float32)
    for j in range(8):
        sj = scores[:, j : j + 1]
        gt = (sj > scores).astype(jnp.float32)
        eq = jnp.where((sj == scores) & (lane > j), 1.0, 0.0)
        rank = rank + gt + eq
    mask = (rank < n_active).astype(jnp.float32)

    w = probs * mask
    w = w / (jnp.sum(w, axis=-1, keepdims=True) + 1e-8)
    sign = jnp.where(lane < n_camp_a, 1.0, -1.0)
    coef_ref[...] = w * sign
    xb_ref[...] = x_ref[...].astype(jnp.bfloat16)


def _fused_body(csel_ref, xb_ref, w1_ref, b1_ref, w2_ref, b2_ref, out_ref, h_ref,
                *, na, no, th, to):
    e = pl.program_id(0)
    p = pl.program_id(1)
    csel = csel_ref[0]

    @pl.when(p < na)
    def _phase_a():
        hc = jnp.dot(xb_ref[...], w1_ref[0].astype(jnp.bfloat16),
                     preferred_element_type=jnp.float32)
        hc = jnp.maximum(hc + b1_ref[0], 0.0)
        val = (hc * csel).astype(jnp.bfloat16)
        for k in range(na):
            @pl.when(p == k)
            def _store(k=k, val=val):
                h_ref[:, k * th:(k + 1) * th] = val

    @pl.when(p >= na)
    def _phase_b():
        t = jnp.dot(h_ref[...], w2_ref[0].astype(jnp.bfloat16),
                    preferred_element_type=jnp.float32)
        t = t + csel * b2_ref[0]
        o = p - na
        for k in range(no):
            @pl.when(o == k)
            def _acc(k=k, t=t):
                @pl.when(e == 0)
                def _init():
                    out_ref[:, k * to:(k + 1) * to] = t

                @pl.when(e > 0)
                def _add():
                    out_ref[:, k * to:(k + 1) * to] += t


def kernel(x, gate_w, gate_b, w1, b1, w2, b2):
    n_tok, d_in = x.shape
    e_num, _, d_hid = w1.shape
    d_out = w2.shape[2]
    n_active = max(1, int(e_num * 0.625))
    n_camp_a = e_num // 2

    gwp = jnp.zeros((d_in, _LANES), jnp.float32).at[:, :e_num].set(gate_w)
    gbp = (
        jnp.full((1, _LANES), -1e30, jnp.float32)
        .at[0, :e_num]
        .set(gate_b.astype(jnp.float32))
    )
    coef, xb = pl.pallas_call(
        functools.partial(_routing_body, n_active=n_active, n_camp_a=n_camp_a),
        out_shape=(
            jax.ShapeDtypeStruct((n_tok, _LANES), jnp.float32),
            jax.ShapeDtypeStruct((n_tok, d_in), jnp.bfloat16),
        ),
    )(x, gwp, gbp)

    csel_all = coef[:, :e_num].T.reshape(e_num, n_tok, 1)

    b1r = b1.reshape(e_num, 1, d_hid).astype(jnp.float32)
    b2r = b2.reshape(e_num, 1, d_out).astype(jnp.float32)

    th = min(256, d_hid)
    to = min(128, d_out)
    na = d_hid // th
    no = d_out // to
    out = pl.pallas_call(
        functools.partial(_fused_body, na=na, no=no, th=th, to=to),
        grid=(e_num, na + no),
        in_specs=[
            pl.BlockSpec((1, n_tok, 1), lambda e, p: (e, 0, 0)),
            pl.BlockSpec((n_tok, d_in), lambda e, p: (0, 0)),
            pl.BlockSpec((1, d_in, th),
                         lambda e, p, _na=na: (e, 0, jnp.minimum(p, _na - 1))),
            pl.BlockSpec((1, 1, th),
                         lambda e, p, _na=na: (e, 0, jnp.minimum(p, _na - 1))),
            pl.BlockSpec((1, d_hid, to),
                         lambda e, p, _na=na: (e, 0, jnp.maximum(p - _na, 0))),
            pl.BlockSpec((1, 1, to),
                         lambda e, p, _na=na: (e, 0, jnp.maximum(p - _na, 0))),
        ],
        out_specs=pl.BlockSpec((n_tok, d_out), lambda e, p: (0, 0)),
        out_shape=jax.ShapeDtypeStruct((n_tok, d_out), jnp.float32),
        scratch_shapes=[pltpu.VMEM((n_tok, d_hid), jnp.bfloat16)],
        compiler_params=pltpu.CompilerParams(
            dimension_semantics=("arbitrary", "arbitrary")
        ),
    )(csel_all, xb, w1, b1r, w2, b2r)
    return out


# speedup vs baseline: 1.4150x; 1.4150x over previous
import functools
import math

import jax
import jax.numpy as jnp
from jax.experimental import pallas as pl
from jax.experimental.pallas import tpu as pltpu

_TEMP = math.e
_LANES = 128


def _routing_body(x_ref, gw_ref, gb_ref, coef_ref, xb_ref, *, n_active, n_camp_a):
    scores = jnp.dot(x_ref[...], gw_ref[...], preferred_element_type=jnp.float32)
    scores = (scores + gb_ref[...]) * (1.0 / _TEMP)
    m = jnp.max(scores, axis=-1, keepdims=True)
    ex = jnp.exp(scores - m)
    probs = ex / jnp.sum(ex, axis=-1, keepdims=True)

    lane = jax.lax.broadcasted_iota(jnp.int32, scores.shape, 1)
    rank = jnp.zeros(scores.shape, jnp.float32)
    for j in range(8):
        sj = scores[:, j : j + 1]
        gt = (sj > scores).astype(jnp.float32)
        eq = jnp.where((sj == scores) & (lane > j), 1.0, 0.0)
        rank = rank + gt + eq
    mask = (rank < n_active).astype(jnp.float32)

    w = probs * mask
    w = w / (jnp.sum(w, axis=-1, keepdims=True) + 1e-8)
    sign = jnp.where(lane < n_camp_a, 1.0, -1.0)
    coef_ref[...] = w * sign
    xb_ref[...] = x_ref[...].astype(jnp.bfloat16)


def _hidden_body(csel_ref, xb_ref, w1_ref, b1_ref, h_ref):
    csel = csel_ref[0]
    hc = jnp.dot(xb_ref[...], w1_ref[0].astype(jnp.bfloat16),
                 preferred_element_type=jnp.float32)
    hc = jnp.maximum(hc + b1_ref[0], 0.0)
    h_ref[0] = (hc * csel).astype(jnp.bfloat16)


def _combine_body(csel_ref, h_ref, w2_ref, b2_ref, out_ref):
    e = pl.program_id(0)
    kh = pl.program_id(1)
    o = pl.program_id(2)

    t = jnp.dot(h_ref[0], w2_ref[0].astype(jnp.bfloat16),
                preferred_element_type=jnp.float32)

    @pl.when(kh == 0)
    def _bias():
        t2 = t + csel_ref[0] * b2_ref[0]

        @pl.when(e == 0)
        def _init():
            out_ref[o] = t2

        @pl.when(e > 0)
        def _acc():
            out_ref[o] += t2

    @pl.when(kh > 0)
    def _acc_kh():
        out_ref[o] += t


def kernel(x, gate_w, gate_b, w1, b1, w2, b2):
    n_tok, d_in = x.shape
    e_num, _, d_hid = w1.shape
    d_out = w2.shape[2]
    n_active = max(1, int(e_num * 0.625))
    n_camp_a = e_num // 2

    gwp = jnp.zeros((d_in, _LANES), jnp.float32).at[:, :e_num].set(gate_w)
    gbp = (
        jnp.full((1, _LANES), -1e30, jnp.float32)
        .at[0, :e_num]
        .set(gate_b.astype(jnp.float32))
    )
    coef, xb = pl.pallas_call(
        functools.partial(_routing_body, n_active=n_active, n_camp_a=n_camp_a),
        out_shape=(
            jax.ShapeDtypeStruct((n_tok, _LANES), jnp.float32),
            jax.ShapeDtypeStruct((n_tok, d_in), jnp.bfloat16),
        ),
    )(x, gwp, gbp)

    csel_all = coef[:, :e_num].T.reshape(e_num, n_tok, 1)

    b1r = b1.reshape(e_num, 1, d_hid).astype(jnp.float32)
    b2r = b2.reshape(e_num, 1, d_out).astype(jnp.float32)

    th = min(512, d_hid)
    h_all = pl.pallas_call(
        _hidden_body,
        grid=(e_num, d_hid // th),
        in_specs=[
            pl.BlockSpec((1, n_tok, 1), lambda e, h: (e, 0, 0)),
            pl.BlockSpec((n_tok, d_in), lambda e, h: (0, 0)),
            pl.BlockSpec((1, d_in, th), lambda e, h: (e, 0, h)),
            pl.BlockSpec((1, 1, th), lambda e, h: (e, 0, h)),
        ],
        out_specs=pl.BlockSpec((1, n_tok, th), lambda e, h: (e, 0, h)),
        out_shape=jax.ShapeDtypeStruct((e_num, n_tok, d_hid), jnp.bfloat16),
        compiler_params=pltpu.CompilerParams(
            dimension_semantics=("arbitrary", "arbitrary")
        ),
    )(csel_all, xb, w1, b1r)

    to = min(512, d_out)
    n_o = d_out // to
    khc = min(2048, d_hid)
    n_kh = d_hid // khc
    out3 = pl.pallas_call(
        _combine_body,
        grid=(e_num, n_kh, n_o),
        in_specs=[
            pl.BlockSpec((1, n_tok, 1), lambda e, kh, o: (e, 0, 0)),
            pl.BlockSpec((1, n_tok, khc), lambda e, kh, o: (e, 0, kh)),
            pl.BlockSpec((1, khc, to), lambda e, kh, o: (e, kh, o)),
            pl.BlockSpec((1, 1, to), lambda e, kh, o: (e, 0, o)),
        ],
        out_specs=pl.BlockSpec((n_o, n_tok, to), lambda e, kh, o: (0, 0, 0)),
        out_shape=jax.ShapeDtypeStruct((n_o, n_tok, to), jnp.float32),
        compiler_params=pltpu.CompilerParams(
            dimension_semantics=("arbitrary", "arbitrary", "arbitrary")
        ),
    )(csel_all, h_all, w2, b2r)
    return out3.transpose(1, 0, 2).reshape(n_tok, d_out)


# speedup vs baseline: 1.4337x; 1.0132x over previous
import functools
import math

import jax
import jax.numpy as jnp
from jax import lax
from jax.experimental import pallas as pl
from jax.experimental.pallas import tpu as pltpu
from jax.experimental.pallas import tpu_sc as plsc

_TEMP = math.e
_LANES = 128


def _scores_body(x_ref, gw_ref, gb_ref, scores_ref, xb_ref):
    scores = jnp.dot(x_ref[...], gw_ref[...], preferred_element_type=jnp.float32)
    scores = (scores + gb_ref[...]) * (1.0 / _TEMP)
    scores_ref[...] = scores[:, :scores_ref.shape[1]]
    xb_ref[...] = x_ref[...].astype(jnp.bfloat16)


def _sc_routing_body(scores_hbm, cselt_hbm, sv, cv, *, e_num, n_active, n_camp_a,
                     tok_per_w, nc):
    wid = lax.axis_index("s") * nc + lax.axis_index("c")
    pltpu.sync_copy(scores_hbm.at[wid], sv)
    for i in range(tok_per_w // 16):
        s = [sv[e, i * 16:(i + 1) * 16] for e in range(e_num)]
        m = s[0]
        for e in range(1, e_num):
            m = jnp.maximum(m, s[e])
        p = [jnp.exp(se - m) for se in s]
        z = p[0]
        for e in range(1, e_num):
            z = z + p[e]
        pn = [pe / z for pe in p]
        w = []
        for e in range(e_num):
            rank = jnp.zeros((16,), jnp.float32)
            for j in range(e_num):
                if j == e:
                    continue
                rank = rank + jnp.where(s[j] > s[e], 1.0, 0.0)
                if j < e:
                    rank = rank + jnp.where(s[j] == s[e], 1.0, 0.0)
            keep = rank < float(n_active)
            w.append(jnp.where(keep, pn[e], 0.0))
        ws = w[0]
        for e in range(1, e_num):
            ws = ws + w[e]
        inv = 1.0 / (ws + 1e-8)
        for e in range(e_num):
            sgn = 1.0 if e < n_camp_a else -1.0
            cv[e, i * 16:(i + 1) * 16] = w[e] * (sgn * inv)
    pltpu.sync_copy(cv, cselt_hbm.at[wid])


def _hidden_body(csel_ref, xb_ref, w1_ref, b1_ref, h_ref):
    csel = csel_ref[0]
    hc = jnp.dot(xb_ref[...], w1_ref[0].astype(jnp.bfloat16),
                 preferred_element_type=jnp.float32)
    hc = jnp.maximum(hc + b1_ref[0], 0.0)
    h_ref[0] = (hc * csel).astype(jnp.bfloat16)


def _combine_body(csel_ref, h_ref, w2_ref, b2_ref, out_ref):
    e = pl.program_id(0)
    kh = pl.program_id(1)
    o = pl.program_id(2)

    t = jnp.dot(h_ref[0], w2_ref[0].astype(jnp.bfloat16),
                preferred_element_type=jnp.float32)

    @pl.when(kh == 0)
    def _bias():
        t2 = t + csel_ref[0] * b2_ref[0]

        @pl.when(e == 0)
        def _init():
            out_ref[o] = t2

        @pl.when(e > 0)
        def _acc():
            out_ref[o] += t2

    @pl.when(kh > 0)
    def _acc_kh():
        out_ref[o] += t


def kernel(x, gate_w, gate_b, w1, b1, w2, b2):
    n_tok, d_in = x.shape
    e_num, _, d_hid = w1.shape
    d_out = w2.shape[2]
    n_active = max(1, int(e_num * 0.625))
    n_camp_a = e_num // 2

    gwp = jnp.zeros((d_in, _LANES), jnp.float32).at[:, :e_num].set(gate_w)
    gbp = (
        jnp.full((1, _LANES), -1e30, jnp.float32)
        .at[0, :e_num]
        .set(gate_b.astype(jnp.float32))
    )
    scores, xb = pl.pallas_call(
        _scores_body,
        out_shape=(
            jax.ShapeDtypeStruct((n_tok, _LANES), jnp.float32),
            jax.ShapeDtypeStruct((n_tok, d_in), jnp.bfloat16),
        ),
    )(x, gwp, gbp)

    info = plsc.get_sparse_core_info()
    nc = info.num_cores
    nw = nc * info.num_subcores
    tok_per_w = n_tok // nw
    sc_route = functools.partial(
        pl.kernel,
        mesh=plsc.VectorSubcoreMesh(core_axis_name="c", subcore_axis_name="s"),
        out_type=jax.ShapeDtypeStruct((nw, e_num, tok_per_w), jnp.float32),
        scratch_types=[
            pltpu.VMEM((e_num, tok_per_w), jnp.float32),
            pltpu.VMEM((e_num, tok_per_w), jnp.float32),
        ],
    )(
        functools.partial(
            _sc_routing_body,
            e_num=e_num,
            n_active=n_active,
            n_camp_a=n_camp_a,
            tok_per_w=tok_per_w,
            nc=nc,
        )
    )
    scorest3 = (
        scores[:, :e_num].T.reshape(e_num, nw, tok_per_w).transpose(1, 0, 2)
    )
    cselt3 = sc_route(scorest3)

    csel_all = cselt3.transpose(1, 0, 2).reshape(e_num, n_tok, 1)

    b1r = b1.reshape(e_num, 1, d_hid).astype(jnp.float32)
    b2r = b2.reshape(e_num, 1, d_out).astype(jnp.float32)

    th = min(512, d_hid)
    h_all = pl.pallas_call(
        _hidden_body,
        grid=(e_num, d_hid // th),
        in_specs=[
            pl.BlockSpec((1, n_tok, 1), lambda e, h: (e, 0, 0)),
            pl.BlockSpec((n_tok, d_in), lambda e, h: (0, 0)),
            pl.BlockSpec((1, d_in, th), lambda e, h: (e, 0, h)),
            pl.BlockSpec((1, 1, th), lambda e, h: (e, 0, h)),
        ],
        out_specs=pl.BlockSpec((1, n_tok, th), lambda e, h: (e, 0, h)),
        out_shape=jax.ShapeDtypeStruct((e_num, n_tok, d_hid), jnp.bfloat16),
        compiler_params=pltpu.CompilerParams(
            dimension_semantics=("arbitrary", "arbitrary")
        ),
    )(csel_all, xb, w1, b1r)

    to = min(512, d_out)
    n_o = d_out // to
    khc = min(2048, d_hid)
    n_kh = d_hid // khc
    out3 = pl.pallas_call(
        _combine_body,
        grid=(e_num, n_kh, n_o),
        in_specs=[
            pl.BlockSpec((1, n_tok, 1), lambda e, kh, o: (e, 0, 0)),
            pl.BlockSpec((1, n_tok, khc), lambda e, kh, o: (e, 0, kh)),
            pl.BlockSpec((1, khc, to), lambda e, kh, o: (e, kh, o)),
            pl.BlockSpec((1, 1, to), lambda e, kh, o: (e, 0, o)),
        ],
        out_specs=pl.BlockSpec((n_o, n_tok, to), lambda e, kh, o: (0, 0, 0)),
        out_shape=jax.ShapeDtypeStruct((n_o, n_tok, to), jnp.float32),
        compiler_params=pltpu.CompilerParams(
            dimension_semantics=("arbitrary", "arbitrary", "arbitrary")
        ),
    )(csel_all, h_all, w2, b2r)
    return out3.transpose(1, 0, 2).reshape(n_tok, d_out)


# speedup vs baseline: 1.5252x; 1.0638x over previous
import functools
import math

import jax
import jax.numpy as jnp
from jax import lax
from jax.experimental import pallas as pl
from jax.experimental.pallas import tpu as pltpu
from jax.experimental.pallas import tpu_sc as plsc

_TEMP = math.e
_LANES = 128


def _scores_body(x_ref, gw_ref, gb_ref, scores_ref, xb_ref):
    scores = jnp.dot(x_ref[...], gw_ref[...], preferred_element_type=jnp.float32)
    scores = (scores + gb_ref[...]) * (1.0 / _TEMP)
    scores_ref[...] = scores[:, :scores_ref.shape[1]]
    xb_ref[...] = x_ref[...].astype(jnp.bfloat16)


def _sc_routing_body(scores_hbm, cselt_hbm, sv, cv, *, e_num, n_active, n_camp_a,
                     tok_per_w, nc):
    wid = lax.axis_index("s") * nc + lax.axis_index("c")
    pltpu.sync_copy(scores_hbm.at[wid], sv)
    for i in range(tok_per_w // 16):
        s = [sv[e, i * 16:(i + 1) * 16] for e in range(e_num)]
        m = s[0]
        for e in range(1, e_num):
            m = jnp.maximum(m, s[e])
        p = [jnp.exp(se - m) for se in s]
        z = p[0]
        for e in range(1, e_num):
            z = z + p[e]
        pn = [pe / z for pe in p]
        w = []
        for e in range(e_num):
            rank = jnp.zeros((16,), jnp.float32)
            for j in range(e_num):
                if j == e:
                    continue
                rank = rank + jnp.where(s[j] > s[e], 1.0, 0.0)
                if j < e:
                    rank = rank + jnp.where(s[j] == s[e], 1.0, 0.0)
            keep = rank < float(n_active)
            w.append(jnp.where(keep, pn[e], 0.0))
        ws = w[0]
        for e in range(1, e_num):
            ws = ws + w[e]
        inv = 1.0 / (ws + 1e-8)
        for e in range(e_num):
            sgn = 1.0 if e < n_camp_a else -1.0
            cv[e, i * 16:(i + 1) * 16] = w[e] * (sgn * inv)
    pltpu.sync_copy(cv, cselt_hbm.at[wid])


def _hidden_body(csel_ref, xb_ref, w1_ref, b1_ref, h_ref):
    csel = csel_ref[0]
    hc = jnp.dot(xb_ref[...], w1_ref[0].astype(jnp.bfloat16),
                 preferred_element_type=jnp.float32)
    hc = jnp.maximum(hc + b1_ref[0], 0.0)
    h_ref[0] = (hc * csel).astype(jnp.bfloat16)


def _combine_body(csel_ref, h_ref, w2_ref, b2_ref, out_ref, *, no, to):
    e = pl.program_id(0)
    kh = pl.program_id(1)
    o = pl.program_id(2)

    t = jnp.dot(h_ref[0], w2_ref[0].astype(jnp.bfloat16),
                preferred_element_type=jnp.float32)
    t = t + jnp.where(kh == 0, 1.0, 0.0) * (csel_ref[0] * b2_ref[0])
    first = (e == 0) & (kh == 0)

    for k in range(no):
        @pl.when(o == k)
        def _col(k=k):
            @pl.when(first)
            def _init():
                out_ref[:, k * to:(k + 1) * to] = t

            @pl.when(jnp.logical_not(first))
            def _acc():
                out_ref[:, k * to:(k + 1) * to] += t


def kernel(x, gate_w, gate_b, w1, b1, w2, b2):
    n_tok, d_in = x.shape
    e_num, _, d_hid = w1.shape
    d_out = w2.shape[2]
    n_active = max(1, int(e_num * 0.625))
    n_camp_a = e_num // 2

    gwp = jnp.zeros((d_in, _LANES), jnp.float32).at[:, :e_num].set(gate_w)
    gbp = (
        jnp.full((1, _LANES), -1e30, jnp.float32)
        .at[0, :e_num]
        .set(gate_b.astype(jnp.float32))
    )
    scores, xb = pl.pallas_call(
        _scores_body,
        out_shape=(
            jax.ShapeDtypeStruct((n_tok, _LANES), jnp.float32),
            jax.ShapeDtypeStruct((n_tok, d_in), jnp.bfloat16),
        ),
    )(x, gwp, gbp)

    info = plsc.get_sparse_core_info()
    nc = info.num_cores
    nw = nc * info.num_subcores
    tok_per_w = n_tok // nw
    sc_route = functools.partial(
        pl.kernel,
        mesh=plsc.VectorSubcoreMesh(core_axis_name="c", subcore_axis_name="s"),
        out_type=jax.ShapeDtypeStruct((nw, e_num, tok_per_w), jnp.float32),
        scratch_types=[
            pltpu.VMEM((e_num, tok_per_w), jnp.float32),
            pltpu.VMEM((e_num, tok_per_w), jnp.float32),
        ],
    )(
        functools.partial(
            _sc_routing_body,
            e_num=e_num,
            n_active=n_active,
            n_camp_a=n_camp_a,
            tok_per_w=tok_per_w,
            nc=nc,
        )
    )
    scorest3 = (
        scores[:, :e_num].T.reshape(e_num, nw, tok_per_w).transpose(1, 0, 2)
    )
    cselt3 = sc_route(scorest3)

    csel_all = cselt3.transpose(1, 0, 2).reshape(e_num, n_tok, 1)

    b1r = b1.reshape(e_num, 1, d_hid).astype(jnp.float32)
    b2r = b2.reshape(e_num, 1, d_out).astype(jnp.float32)

    th = min(512, d_hid)
    h_all = pl.pallas_call(
        _hidden_body,
        grid=(e_num, d_hid // th),
        in_specs=[
            pl.BlockSpec((1, n_tok, 1), lambda e, h: (e, 0, 0)),
            pl.BlockSpec((n_tok, d_in), lambda e, h: (0, 0)),
            pl.BlockSpec((1, d_in, th), lambda e, h: (e, 0, h)),
            pl.BlockSpec((1, 1, th), lambda e, h: (e, 0, h)),
        ],
        out_specs=pl.BlockSpec((1, n_tok, th), lambda e, h: (e, 0, h)),
        out_shape=jax.ShapeDtypeStruct((e_num, n_tok, d_hid), jnp.bfloat16),
        compiler_params=pltpu.CompilerParams(
            dimension_semantics=("arbitrary", "arbitrary")
        ),
    )(csel_all, xb, w1, b1r)

    to = min(512, d_out)
    n_o = d_out // to
    khc = min(2048, d_hid)
    n_kh = d_hid // khc
    out = pl.pallas_call(
        functools.partial(_combine_body, no=n_o, to=to),
        grid=(e_num, n_kh, n_o),
        in_specs=[
            pl.BlockSpec((1, n_tok, 1), lambda e, kh, o: (e, 0, 0)),
            pl.BlockSpec((1, n_tok, khc), lambda e, kh, o: (e, 0, kh)),
            pl.BlockSpec((1, khc, to), lambda e, kh, o: (e, kh, o)),
            pl.BlockSpec((1, 1, to), lambda e, kh, o: (e, 0, o)),
        ],
        out_specs=pl.BlockSpec((n_tok, d_out), lambda e, kh, o: (0, 0)),
        out_shape=jax.ShapeDtypeStruct((n_tok, d_out), jnp.float32),
        compiler_params=pltpu.CompilerParams(
            dimension_semantics=("arbitrary", "arbitrary", "arbitrary")
        ),
    )(csel_all, h_all, w2, b2r)
    return out
